# baseline (speedup 1.0000x reference)
import functools

import jax
import jax.numpy as jnp
from jax import lax
from jax.experimental import pallas as pl
from jax.experimental.pallas import tpu as pltpu
from jax.experimental.pallas import tpu_sc as plsc

B = 4096
T = 48
EMB = 32
H = 1024
N_TS_CONT = 8
N_TS_CAT = 4
N_ST_CONT = 10
N_ST_CAT = 6
VOCAB = 1000
EPS = 1e-5

K_CAT = N_TS_CAT * T + N_ST_CAT
K_PAD = 200
ZROW = 10 * VOCAB
D_CAT = K_CAT * EMB
NG4 = K_PAD * EMB // 128
KC = T * N_TS_CONT + N_ST_CONT
KCP = 512

NW = 32
G = 128

BM = 512
NB = B // BM
NH = 2
BH = B // NH
NBH = BH // BM

R_HALF = BH * K_PAD
R_W = R_HALF // NW
N_G = 20
N_CHUNK = R_W // (N_G * G)


def _gather_body(tab_hbm, idx_hbm, out_hbm, idx_v, rows_v, sem):
    chunk_rows = N_G * G
    wid = lax.axis_index("s") * 2 + lax.axis_index("c")
    pltpu.sync_copy(idx_hbm.at[wid], idx_v)

    def chunk(c, carry):
        row0 = wid * R_W + c * chunk_rows
        descs = [
            pltpu.async_copy(
                tab_hbm.at[idx_v.at[c * N_G + j]],
                rows_v.at[pl.ds(j * G, G)],
                sem,
            )
            for j in range(N_G)
        ]
        for d in descs:
            d.wait()
        pltpu.sync_copy(rows_v, out_hbm.at[pl.ds(row0, chunk_rows)])
        return carry

    lax.fori_loop(0, N_CHUNK, chunk, 0)


@functools.lru_cache(maxsize=None)
def _gather_fn():
    return pl.kernel(
        _gather_body,
        out_type=jax.ShapeDtypeStruct((R_HALF, EMB), jnp.float32),
        mesh=plsc.VectorSubcoreMesh(core_axis_name="c", subcore_axis_name="s"),
        scratch_types=[
            pltpu.VMEM((R_W // G, G), jnp.int32),
            pltpu.VMEM((N_G * G, EMB), jnp.float32),
            pltpu.SemaphoreType.DMA,
        ],
        compiler_params=pltpu.CompilerParams(use_tc_tiling_on_sc=False),
    )


def _mm_body(x_ref, xc_ref, w_ref, wc_ref, b_ref, o_ref):
    def g_step(g, acc):
        return acc + jnp.dot(x_ref[g].astype(jnp.bfloat16), w_ref[g],
                             preferred_element_type=jnp.float32)

    acc = lax.fori_loop(0, NG4, g_step, jnp.zeros((BM, H), jnp.float32))
    acc += jnp.dot(xc_ref[...], wc_ref[...],
                   preferred_element_type=jnp.float32)
    o_ref[...] = acc + b_ref[...]


def _chain_body(h0_ref, w_ref, bh_ref, g_ref, be_ref, o_ref, hs, stats, acc):
    l = pl.program_id(0)
    j = pl.program_id(1)

    def _accumulate(r):
        acc[0:1, :] += jnp.sum(r, axis=0, keepdims=True)
        acc[1:2, :] += jnp.sum(r * r, axis=0, keepdims=True)

    @pl.when((l == 0) & (j == 0))
    def _():
        acc[...] = jnp.zeros_like(acc)

    @pl.when((l > 0) & (j == 0))
    def _():
        mu = acc[0:1, :] * (1.0 / B)
        var = acc[1:2, :] * (1.0 / B) - mu * mu
        stats[0:1, :] = mu
        stats[1:2, :] = lax.rsqrt(var + EPS)
        acc[...] = jnp.zeros_like(acc)

    @pl.when((l == 0) & (j > 0))
    def _():
        blk = h0_ref[...]
        hs[pl.ds((j - 1) * BM, BM), :] = blk
        _accumulate(jnp.maximum(blk, 0.0))

    @pl.when((l > 0) & (j > 0))
    def _():
        b = (j - 1) * BM
        r = jnp.maximum(hs[pl.ds(b, BM), :], 0.0)
        hn = (r - stats[0:1, :]) * stats[1:2, :] * g_ref[0] + be_ref[0]
        h2 = jnp.dot(hn.astype(jnp.bfloat16), w_ref[0],
                     preferred_element_type=jnp.float32) + bh_ref[0]

        @pl.when(l < 3)
        def _():
            hs[pl.ds(b, BM), :] = h2
            _accumulate(jnp.maximum(h2, 0.0))

        @pl.when(l == 3)
        def _():
            o_ref[...] = h2


def _first_layer(x4, xc, W4, W0c, b0r):
    return pl.pallas_call(
        _mm_body,
        grid=(NBH,),
        in_specs=[
            pl.BlockSpec((NG4, BM, 128), lambda i: (0, i, 0)),
            pl.BlockSpec((BM, KCP), lambda i: (i, 0)),
            pl.BlockSpec((NG4, 128, H), lambda i: (0, 0, 0)),
            pl.BlockSpec((KCP, H), lambda i: (0, 0)),
            pl.BlockSpec((1, H), lambda i: (0, 0)),
        ],
        out_specs=pl.BlockSpec((BM, H), lambda i: (i, 0)),
        out_shape=jax.ShapeDtypeStruct((BH, H), jnp.float32),
        compiler_params=pltpu.CompilerParams(
            dimension_semantics=("arbitrary",),
        ),
    )(x4, xc, W4, W0c, b0r)


def _chain(h0, Wh, bh, gamma, beta):
    blk = lambda l, j: (jnp.maximum(j, 1) - 1, 0)
    lyr = lambda l, j: (jnp.maximum(l, 1) - 1, 0, 0)
    return pl.pallas_call(
        _chain_body,
        grid=(4, NB + 1),
        in_specs=[
            pl.BlockSpec((BM, H), blk),
            pl.BlockSpec((1, H, H), lyr),
            pl.BlockSpec((1, 1, H), lyr),
            pl.BlockSpec((1, 1, H), lyr),
            pl.BlockSpec((1, 1, H), lyr),
        ],
        out_specs=pl.BlockSpec((BM, H), blk),
        out_shape=jax.ShapeDtypeStruct((B, H), jnp.float32),
        scratch_shapes=[
            pltpu.VMEM((B, H), jnp.float32),
            pltpu.VMEM((8, H), jnp.float32),
            pltpu.VMEM((8, H), jnp.float32),
        ],
        compiler_params=pltpu.CompilerParams(
            dimension_semantics=("arbitrary", "arbitrary"),
        ),
    )(h0, Wh, bh.reshape(3, 1, H), gamma.reshape(3, 1, H),
      beta.reshape(3, 1, H))


def kernel(ts_cont_feats, ts_cat_feats, static_cont_feats, static_cat_feats,
           ts_tables, static_tables, W0, b0, Wh, bh, gamma, beta):
    idx_ts = ts_cat_feats.astype(jnp.int32).transpose(0, 2, 1) \
        + (jnp.arange(N_TS_CAT, dtype=jnp.int32) * VOCAB)[None, :, None]
    idx_st = static_cat_feats.astype(jnp.int32) \
        + N_TS_CAT * VOCAB + jnp.arange(N_ST_CAT, dtype=jnp.int32) * VOCAB
    idx = jnp.concatenate(
        [idx_ts.reshape(B, N_TS_CAT * T), idx_st,
         jnp.full((B, K_PAD - K_CAT), ZROW, jnp.int32)], axis=1)
    tab = jnp.concatenate(
        [ts_tables.reshape(N_TS_CAT * VOCAB, EMB),
         static_tables.reshape(N_ST_CAT * VOCAB, EMB),
         jnp.zeros((8, EMB), jnp.float32)], axis=0)

    xc = jnp.concatenate(
        [ts_cont_feats.astype(jnp.float32).reshape(B, T * N_TS_CONT),
         static_cont_feats.astype(jnp.float32)], axis=1)
    xc = jnp.pad(xc, ((0, 0), (0, KCP - KC))).astype(jnp.bfloat16)
    W4 = jnp.pad(W0[:D_CAT], ((0, K_PAD * EMB - D_CAT), (0, 0))) \
        .astype(jnp.bfloat16).reshape(NG4, 128, H)
    W0c = jnp.pad(W0[D_CAT:], ((0, KCP - KC), (0, 0))).astype(jnp.bfloat16)
    b0r = b0.reshape(1, H)

    gather = _gather_fn()
    x4s = []
    for hlf in range(NH):
        idx_h = idx[hlf * BH:(hlf + 1) * BH] \
            .reshape(BH, NG4, 4).transpose(1, 0, 2).reshape(NW, R_W // G, G)
        x4s.append(gather(tab, idx_h).reshape(NG4, BH, 128))
    halves = [
        _first_layer(x4s[hlf], xc[hlf * BH:(hlf + 1) * BH], W4, W0c, b0r)
        for hlf in range(NH)
    ]
    h0 = jnp.concatenate(halves, axis=0)
    return _chain(h0, Wh.astype(jnp.bfloat16), bh, gamma, beta)


# speedup vs baseline: 1.5175x; 1.5175x over previous
import functools

import jax
import jax.numpy as jnp
from jax import lax
from jax.experimental import pallas as pl
from jax.experimental.pallas import tpu as pltpu
from jax.experimental.pallas import tpu_sc as plsc

B = 4096
T = 48
EMB = 32
H = 1024
N_TS_CONT = 8
N_TS_CAT = 4
N_ST_CONT = 10
N_ST_CAT = 6
VOCAB = 1000
EPS = 1e-5

K_CAT = N_TS_CAT * T + N_ST_CAT
K_PAD = 200
ZROW = 10 * VOCAB
D_CAT = K_CAT * EMB
NG4 = K_PAD * EMB // 128
KC = T * N_TS_CONT + N_ST_CONT
KCP = 512

NW = 32
G = 128

BM = 512
NB = B // BM
NH = 2
BH = B // NH
NBH = BH // BM

R_HALF = BH * K_PAD
R_W = R_HALF // NW
N_G = 10
N_CHUNK = R_W // (N_G * G)
CHUNK_ROWS = N_G * G


def _gather_body(tab_hbm, idx_hbm, out_hbm, idx_v, rows_a, rows_b,
                 sem, wsem_a, wsem_b):
    wid = lax.axis_index("s") * 2 + lax.axis_index("c")
    pltpu.sync_copy(idx_hbm.at[wid], idx_v)
    bufs = (rows_a, rows_b)
    wsems = (wsem_a, wsem_b)

    def chunk_pair(cp, carry):
        for par in range(2):
            c = cp * 2 + par
            buf, ws = bufs[par], wsems[par]
            row0 = wid * R_W + c * CHUNK_ROWS

            @pl.when(cp > 0)
            def _():
                pltpu.make_async_copy(
                    buf, out_hbm.at[pl.ds(0, CHUNK_ROWS)], ws).wait()

            descs = [
                pltpu.async_copy(
                    tab_hbm.at[idx_v.at[c * N_G + j]],
                    buf.at[pl.ds(j * G, G)],
                    sem,
                )
                for j in range(N_G)
            ]
            for d in descs:
                d.wait()
            pltpu.async_copy(buf, out_hbm.at[pl.ds(row0, CHUNK_ROWS)], ws)
        return carry

    lax.fori_loop(0, N_CHUNK // 2, chunk_pair, 0)
    for par in range(2):
        pltpu.make_async_copy(
            bufs[par], out_hbm.at[pl.ds(0, CHUNK_ROWS)], wsems[par]).wait()


@functools.lru_cache(maxsize=None)
def _gather_fn():
    return pl.kernel(
        _gather_body,
        out_type=jax.ShapeDtypeStruct((R_HALF, EMB), jnp.float32),
        mesh=plsc.VectorSubcoreMesh(core_axis_name="c", subcore_axis_name="s"),
        scratch_types=[
            pltpu.VMEM((R_W // G, G), jnp.int32),
            pltpu.VMEM((CHUNK_ROWS, EMB), jnp.float32),
            pltpu.VMEM((CHUNK_ROWS, EMB), jnp.float32),
            pltpu.SemaphoreType.DMA,
            pltpu.SemaphoreType.DMA,
            pltpu.SemaphoreType.DMA,
        ],
        compiler_params=pltpu.CompilerParams(use_tc_tiling_on_sc=False),
    )


def _mm_body(x_ref, xc_ref, w_ref, wc_ref, b_ref, o_ref):
    xx = jnp.concatenate(
        [x_ref[g].astype(jnp.bfloat16) for g in range(NG4)], axis=1)
    acc = jnp.dot(xx, w_ref[...], preferred_element_type=jnp.float32)
    acc += jnp.dot(xc_ref[...], wc_ref[...],
                   preferred_element_type=jnp.float32)
    o_ref[...] = acc + b_ref[...]


def _chain_body(h0a_ref, h0b_ref, w_ref, bh_ref, g_ref, be_ref, o_ref,
                hs, stats, acc):
    l = pl.program_id(0)
    j = pl.program_id(1)

    def _accumulate(r):
        acc[0:1, :] += jnp.sum(r, axis=0, keepdims=True)
        acc[1:2, :] += jnp.sum(r * r, axis=0, keepdims=True)

    @pl.when((l == 0) & (j == 0))
    def _():
        acc[...] = jnp.zeros_like(acc)

    @pl.when((l > 0) & (j == 0))
    def _():
        mu = acc[0:1, :] * (1.0 / B)
        var = acc[1:2, :] * (1.0 / B) - mu * mu
        sc = lax.rsqrt(var + EPS) * g_ref[0]
        stats[0:1, :] = sc
        stats[1:2, :] = be_ref[0] - mu * sc
        acc[...] = jnp.zeros_like(acc)

    @pl.when((l == 0) & (j > 0))
    def _():
        blk = jnp.where(j <= NBH, h0a_ref[...], h0b_ref[...])
        r = jnp.maximum(blk, 0.0)
        hs[pl.ds((j - 1) * BM, BM), :] = r
        _accumulate(r)

    @pl.when((l > 0) & (j > 0))
    def _():
        b = (j - 1) * BM
        r = hs[pl.ds(b, BM), :]
        hn = r * stats[0:1, :] + stats[1:2, :]
        h2 = jnp.dot(hn.astype(jnp.bfloat16), w_ref[0],
                     preferred_element_type=jnp.float32) + bh_ref[0]

        @pl.when(l < 3)
        def _():
            r2 = jnp.maximum(h2, 0.0)
            hs[pl.ds(b, BM), :] = r2
            _accumulate(r2)

        @pl.when(l == 3)
        def _():
            o_ref[...] = h2


def _first_layer(x4, xc, W4, W0c, b0r):
    return pl.pallas_call(
        _mm_body,
        grid=(NBH,),
        in_specs=[
            pl.BlockSpec((NG4, BM, 128), lambda i: (0, i, 0)),
            pl.BlockSpec((BM, KCP), lambda i: (i, 0)),
            pl.BlockSpec((K_PAD * EMB, H), lambda i: (0, 0)),
            pl.BlockSpec((KCP, H), lambda i: (0, 0)),
            pl.BlockSpec((1, H), lambda i: (0, 0)),
        ],
        out_specs=pl.BlockSpec((BM, H), lambda i: (i, 0)),
        out_shape=jax.ShapeDtypeStruct((BH, H), jnp.float32),
        compiler_params=pltpu.CompilerParams(
            dimension_semantics=("arbitrary",),
        ),
    )(x4, xc, W4, W0c, b0r)


def _chain(h0a, h0b, Wh, bh, gamma, beta):
    ha = lambda l, j: (jnp.where(l == 0, jnp.clip(j - 1, 0, NBH - 1), 0), 0)
    hb = lambda l, j: (jnp.where(l == 0, jnp.clip(j - 1 - NBH, 0, NBH - 1),
                                 0), 0)
    om = lambda l, j: (jnp.where(l == 3, jnp.maximum(j, 1) - 1, 0), 0)
    lyr = lambda l, j: (jnp.maximum(l, 1) - 1, 0, 0)
    return pl.pallas_call(
        _chain_body,
        grid=(4, NB + 1),
        in_specs=[
            pl.BlockSpec((BM, H), ha),
            pl.BlockSpec((BM, H), hb),
            pl.BlockSpec((1, H, H), lyr),
            pl.BlockSpec((1, 1, H), lyr),
            pl.BlockSpec((1, 1, H), lyr),
            pl.BlockSpec((1, 1, H), lyr),
        ],
        out_specs=pl.BlockSpec((BM, H), om),
        out_shape=jax.ShapeDtypeStruct((B, H), jnp.float32),
        scratch_shapes=[
            pltpu.VMEM((B, H), jnp.float32),
            pltpu.VMEM((8, H), jnp.float32),
            pltpu.VMEM((8, H), jnp.float32),
        ],
        compiler_params=pltpu.CompilerParams(
            dimension_semantics=("arbitrary", "arbitrary"),
        ),
    )(h0a, h0b, Wh, bh.reshape(3, 1, H), gamma.reshape(3, 1, H),
      beta.reshape(3, 1, H))


def kernel(ts_cont_feats, ts_cat_feats, static_cont_feats, static_cat_feats,
           ts_tables, static_tables, W0, b0, Wh, bh, gamma, beta):
    idx_ts = ts_cat_feats.astype(jnp.int32).transpose(0, 2, 1) \
        + (jnp.arange(N_TS_CAT, dtype=jnp.int32) * VOCAB)[None, :, None]
    idx_st = static_cat_feats.astype(jnp.int32) \
        + N_TS_CAT * VOCAB + jnp.arange(N_ST_CAT, dtype=jnp.int32) * VOCAB
    idx = jnp.concatenate(
        [idx_ts.reshape(B, N_TS_CAT * T), idx_st,
         jnp.full((B, K_PAD - K_CAT), ZROW, jnp.int32)], axis=1)
    tab = jnp.concatenate(
        [ts_tables.reshape(N_TS_CAT * VOCAB, EMB),
         static_tables.reshape(N_ST_CAT * VOCAB, EMB),
         jnp.zeros((8, EMB), jnp.float32)], axis=0)

    xc = jnp.concatenate(
        [ts_cont_feats.astype(jnp.float32).reshape(B, T * N_TS_CONT),
         static_cont_feats.astype(jnp.float32)], axis=1)
    xc = jnp.pad(xc, ((0, 0), (0, KCP - KC))).astype(jnp.bfloat16)
    W4 = jnp.pad(W0[:D_CAT], ((0, K_PAD * EMB - D_CAT), (0, 0))) \
        .astype(jnp.bfloat16)
    W0c = jnp.pad(W0[D_CAT:], ((0, KCP - KC), (0, 0))).astype(jnp.bfloat16)
    b0r = b0.reshape(1, H)

    gather = _gather_fn()
    x4s = []
    for hlf in range(NH):
        idx_h = idx[hlf * BH:(hlf + 1) * BH] \
            .reshape(BH, NG4, 4).transpose(1, 0, 2).reshape(NW, R_W // G, G)
        x4s.append(gather(tab, idx_h).reshape(NG4, BH, 128))
    halves = [
        _first_layer(x4s[hlf], xc[hlf * BH:(hlf + 1) * BH], W4, W0c, b0r)
        for hlf in range(NH)
    ]
    return _chain(halves[0], halves[1], Wh.astype(jnp.bfloat16),
                  bh, gamma, beta)


# speedup vs baseline: 1.7001x; 1.1203x over previous
import functools

import jax
import jax.numpy as jnp
from jax import lax
from jax.experimental import pallas as pl
from jax.experimental.pallas import tpu as pltpu
from jax.experimental.pallas import tpu_sc as plsc

B = 4096
T = 48
EMB = 32
H = 1024
N_TS_CONT = 8
N_TS_CAT = 4
N_ST_CONT = 10
N_ST_CAT = 6
VOCAB = 1000
EPS = 1e-5

K_CAT = N_TS_CAT * T + N_ST_CAT
K_PAD = 200
ZROW = 10 * VOCAB
D_CAT = K_CAT * EMB
NG4 = K_PAD * EMB // 128
KC = T * N_TS_CONT + N_ST_CONT
KCP = 512

NW = 32
G = 128

BM = 512
NB = B // BM
NH = 2
BH = B // NH
NBH = BH // BM

R_HALF = BH * K_PAD
R_W = R_HALF // NW
N_G = 20
N_CHUNK = R_W // (N_G * G)
CHUNK_ROWS = N_G * G


def _gather_body(tab_hbm, idx_hbm, sidx_hbm, out_hbm, idx_v, sidx_v, rows_v,
                 sem, wsem):
    wid = lax.axis_index("s") * 2 + lax.axis_index("c")
    pltpu.sync_copy(idx_hbm.at[wid], idx_v)
    pltpu.sync_copy(sidx_hbm.at[wid], sidx_v)

    def chunk(c, carry):
        descs = [
            pltpu.async_copy(
                tab_hbm.at[idx_v.at[c * N_G + j]],
                rows_v.at[pl.ds(j * G, G)],
                sem,
            )
            for j in range(N_G)
        ]
        for d in descs:
            d.wait()
        wdescs = [
            pltpu.async_copy(
                rows_v.at[pl.ds(j * G, G)],
                out_hbm.at[sidx_v.at[c * N_G + j]],
                wsem,
            )
            for j in range(N_G)
        ]
        for d in wdescs:
            d.wait()
        return carry

    lax.fori_loop(0, N_CHUNK, chunk, 0)


@functools.lru_cache(maxsize=None)
def _gather_fn():
    return pl.kernel(
        _gather_body,
        out_type=jax.ShapeDtypeStruct((R_HALF, EMB), jnp.float32),
        mesh=plsc.VectorSubcoreMesh(core_axis_name="c", subcore_axis_name="s"),
        scratch_types=[
            pltpu.VMEM((R_W // G, G), jnp.int32),
            pltpu.VMEM((R_W // G, G), jnp.int32),
            pltpu.VMEM((CHUNK_ROWS, EMB), jnp.float32),
            pltpu.SemaphoreType.DMA,
            pltpu.SemaphoreType.DMA,
        ],
        compiler_params=pltpu.CompilerParams(use_tc_tiling_on_sc=False),
    )


def _mm_body(x_ref, xc_ref, w_ref, wc_ref, b_ref, o_ref):
    xx = jnp.concatenate(
        [x_ref[g].astype(jnp.bfloat16) for g in range(NG4)], axis=1)
    acc = jnp.dot(xx, w_ref[...], preferred_element_type=jnp.float32)
    acc += jnp.dot(xc_ref[...], wc_ref[...],
                   preferred_element_type=jnp.float32)
    o_ref[...] = acc + b_ref[...]


def _chain_body(h0a_ref, h0b_ref, w_ref, bh_ref, g_ref, be_ref, o_ref,
                hs, stats, acc):
    l = pl.program_id(0)
    j = pl.program_id(1)

    def _accumulate(r):
        acc[0:1, :] += jnp.sum(r, axis=0, keepdims=True)
        acc[1:2, :] += jnp.sum(r * r, axis=0, keepdims=True)

    @pl.when((l == 0) & (j == 0))
    def _():
        acc[...] = jnp.zeros_like(acc)

    @pl.when((l > 0) & (j == 0))
    def _():
        mu = acc[0:1, :] * (1.0 / B)
        var = acc[1:2, :] * (1.0 / B) - mu * mu
        sc = lax.rsqrt(var + EPS) * g_ref[0]
        stats[0:1, :] = sc
        stats[1:2, :] = be_ref[0] - mu * sc
        acc[...] = jnp.zeros_like(acc)

    @pl.when((l == 0) & (j > 0))
    def _():
        blk = jnp.where(j <= NBH, h0a_ref[...], h0b_ref[...])
        r = jnp.maximum(blk, 0.0)
        hs[pl.ds((j - 1) * BM, BM), :] = r
        _accumulate(r)

    @pl.when((l > 0) & (j > 0))
    def _():
        b = (j - 1) * BM
        r = hs[pl.ds(b, BM), :]
        hn = r * stats[0:1, :] + stats[1:2, :]
        h2 = jnp.dot(hn.astype(jnp.bfloat16), w_ref[0],
                     preferred_element_type=jnp.float32) + bh_ref[0]

        @pl.when(l < 3)
        def _():
            r2 = jnp.maximum(h2, 0.0)
            hs[pl.ds(b, BM), :] = r2
            _accumulate(r2)

        @pl.when(l == 3)
        def _():
            o_ref[...] = h2


def _first_layer(x4, xc, W4, W0c, b0r):
    return pl.pallas_call(
        _mm_body,
        grid=(NBH,),
        in_specs=[
            pl.BlockSpec((NG4, BM, 128), lambda i: (0, i, 0)),
            pl.BlockSpec((BM, KCP), lambda i: (i, 0)),
            pl.BlockSpec((K_PAD * EMB, H), lambda i: (0, 0)),
            pl.BlockSpec((KCP, H), lambda i: (0, 0)),
            pl.BlockSpec((1, H), lambda i: (0, 0)),
        ],
        out_specs=pl.BlockSpec((BM, H), lambda i: (i, 0)),
        out_shape=jax.ShapeDtypeStruct((BH, H), jnp.float32),
        compiler_params=pltpu.CompilerParams(
            dimension_semantics=("arbitrary",),
        ),
    )(x4, xc, W4, W0c, b0r)


def _chain(h0a, h0b, Wh, bh, gamma, beta):
    ha = lambda l, j: (jnp.where(l == 0, jnp.clip(j - 1, 0, NBH - 1), 0), 0)
    hb = lambda l, j: (jnp.where(l == 0, jnp.clip(j - 1 - NBH, 0, NBH - 1),
                                 0), 0)
    om = lambda l, j: (jnp.where(l == 3, jnp.maximum(j, 1) - 1, 0), 0)
    lyr = lambda l, j: (jnp.maximum(l, 1) - 1, 0, 0)
    return pl.pallas_call(
        _chain_body,
        grid=(4, NB + 1),
        in_specs=[
            pl.BlockSpec((BM, H), ha),
            pl.BlockSpec((BM, H), hb),
            pl.BlockSpec((1, H, H), lyr),
            pl.BlockSpec((1, 1, H), lyr),
            pl.BlockSpec((1, 1, H), lyr),
            pl.BlockSpec((1, 1, H), lyr),
        ],
        out_specs=pl.BlockSpec((BM, H), om),
        out_shape=jax.ShapeDtypeStruct((B, H), jnp.float32),
        scratch_shapes=[
            pltpu.VMEM((B, H), jnp.float32),
            pltpu.VMEM((8, H), jnp.float32),
            pltpu.VMEM((8, H), jnp.float32),
        ],
        compiler_params=pltpu.CompilerParams(
            dimension_semantics=("arbitrary", "arbitrary"),
        ),
    )(h0a, h0b, Wh, bh.reshape(3, 1, H), gamma.reshape(3, 1, H),
      beta.reshape(3, 1, H))


def kernel(ts_cont_feats, ts_cat_feats, static_cont_feats, static_cat_feats,
           ts_tables, static_tables, W0, b0, Wh, bh, gamma, beta):
    idx_ts = ts_cat_feats.astype(jnp.int32).transpose(0, 2, 1) \
        + (jnp.arange(N_TS_CAT, dtype=jnp.int32) * VOCAB)[None, :, None]
    idx_st = static_cat_feats.astype(jnp.int32) \
        + N_TS_CAT * VOCAB + jnp.arange(N_ST_CAT, dtype=jnp.int32) * VOCAB
    idx = jnp.concatenate(
        [idx_ts.reshape(B, N_TS_CAT * T), idx_st,
         jnp.full((B, K_PAD - K_CAT), ZROW, jnp.int32)], axis=1)
    tab = jnp.concatenate(
        [ts_tables.reshape(N_TS_CAT * VOCAB, EMB),
         static_tables.reshape(N_ST_CAT * VOCAB, EMB),
         jnp.zeros((8, EMB), jnp.float32)], axis=0)

    xc = jnp.concatenate(
        [ts_cont_feats.astype(jnp.float32).reshape(B, T * N_TS_CONT),
         static_cont_feats.astype(jnp.float32)], axis=1)
    xc = jnp.pad(xc, ((0, 0), (0, KCP - KC))).astype(jnp.bfloat16)
    W4 = jnp.pad(W0[:D_CAT], ((0, K_PAD * EMB - D_CAT), (0, 0))) \
        .astype(jnp.bfloat16)
    W0c = jnp.pad(W0[D_CAT:], ((0, KCP - KC), (0, 0))).astype(jnp.bfloat16)
    b0r = b0.reshape(1, H)

    gather = _gather_fn()
    r = jnp.arange(R_HALF, dtype=jnp.int32)
    bb, kk = r // K_PAD, r % K_PAD
    sidx = ((kk // 4) * (4 * BH) + bb * 4 + kk % 4).reshape(NW, R_W // G, G)
    x4s = []
    for hlf in range(NH):
        idx_h = idx[hlf * BH:(hlf + 1) * BH].reshape(NW, R_W // G, G)
        x4s.append(gather(tab, idx_h, sidx).reshape(NG4, BH, 128))
    halves = [
        _first_layer(x4s[hlf], xc[hlf * BH:(hlf + 1) * BH], W4, W0c, b0r)
        for hlf in range(NH)
    ]
    return _chain(halves[0], halves[1], Wh.astype(jnp.bfloat16),
                  bh, gamma, beta)
